# baseline (speedup 1.0000x reference)
import functools

import jax
import jax.numpy as jnp
from jax import lax
from jax.experimental import pallas as pl
from jax.experimental.pallas import tpu as pltpu
from jax.experimental.pallas import tpu_sc as plsc

_N = 18432
_K = 1024
_D = 64
_BLK = 3072
_COMMIT = 0.25
_NA = 12288
_NB = _N - _NA


def _tc_body(xt_ref, wt_ref, w2_ref, idx_ref, loss_ref):
    i = pl.program_id(0)
    xt = xt_ref[...]
    wt = wt_ref[...]
    mmt = lax.dot_general(wt, xt, (((0,), (0,)), ((), ())),
                          preferred_element_type=jnp.float32)
    z2 = jnp.sum(xt * xt, axis=0, keepdims=True)
    w2 = w2_ref[...]
    dist = (z2 + w2) - mmt
    minval = jnp.min(dist, axis=0, keepdims=True)
    idx = jnp.argmin(dist, axis=0).reshape(1, _BLK)
    idx_ref[...] = idx[None]

    @pl.when(i == 0)
    def _init():
        loss_ref[...] = jnp.zeros_like(loss_ref)

    loss_ref[...] = loss_ref[...] + jnp.sum(minval)


def _make_tc_call(n_tok, block_off):
    return pl.pallas_call(
        _tc_body,
        grid=(n_tok // _BLK,),
        in_specs=[
            pl.BlockSpec((_D, _BLK), lambda i: (0, i + block_off)),
            pl.BlockSpec((_D, _K), lambda i: (0, 0)),
            pl.BlockSpec((_K, 1), lambda i: (0, 0)),
        ],
        out_specs=[
            pl.BlockSpec((1, 1, _BLK), lambda i: (i, 0, 0)),
            pl.BlockSpec((1, 1), lambda i: (0, 0)),
        ],
        out_shape=[
            jax.ShapeDtypeStruct((n_tok // _BLK, 1, _BLK), jnp.int32),
            jax.ShapeDtypeStruct((1, 1), jnp.float32),
        ],
    )


_tc_half = [_make_tc_call(_NA, 0), _make_tc_call(_NB, _NA // _BLK)]

_DP = 128


@functools.cache
def _make_sc_gather(n_tok):
    info = plsc.get_sparse_core_info()
    nc, ns = info.num_cores, info.num_subcores
    nw = nc * ns
    bpw = n_tok // nw
    ch = 96
    n_ch = bpw // ch
    assert bpw % ch == 0 and n_tok % nw == 0
    mesh = plsc.VectorSubcoreMesh(core_axis_name="c", subcore_axis_name="s")

    @functools.partial(
        pl.kernel,
        mesh=mesh,
        out_type=jax.ShapeDtypeStruct((n_tok, _DP), jnp.float32),
        scratch_types=[
            pltpu.VMEM((bpw,), jnp.int32),
            pltpu.VMEM((bpw, _DP), jnp.float32),
            pltpu.SemaphoreType.DMA,
        ],
    )
    def gather_k(idx_hbm, table_hbm, out_hbm, idx_v, rows_v, sem):
        wid = lax.axis_index("s") * nc + lax.axis_index("c")
        base = wid * bpw
        pltpu.sync_copy(idx_hbm.at[pl.ds(base, bpw)], idx_v)
        copies = []
        for j in range(n_ch):
            copies.append(pltpu.async_copy(
                table_hbm.at[idx_v.at[pl.ds(j * ch, ch)]],
                rows_v.at[pl.ds(j * ch, ch)],
                sem,
            ))
        for cp in copies:
            cp.wait()
        pltpu.sync_copy(rows_v, out_hbm.at[pl.ds(base, bpw)])

    return gather_k


def _tp_body_first(q_ref, out_ref):
    out_ref[...] = q_ref[...][:, :_D].T


def _tp_body_second(q_ref, alias_ref, out_ref):
    del alias_ref
    out_ref[...] = q_ref[...][:, :_D].T


_tp_first = pl.pallas_call(
    _tp_body_first,
    grid=(_NA // _BLK,),
    in_specs=[pl.BlockSpec((_BLK, _DP), lambda j: (j, 0))],
    out_specs=pl.BlockSpec((_D, _BLK), lambda j: (0, j)),
    out_shape=jax.ShapeDtypeStruct((_D, _N), jnp.float32),
)

_tp_second = pl.pallas_call(
    _tp_body_second,
    grid=(_NB // _BLK,),
    in_specs=[
        pl.BlockSpec((_BLK, _DP), lambda j: (j, 0)),
        pl.BlockSpec(memory_space=pl.ANY),
    ],
    out_specs=pl.BlockSpec((_D, _BLK), lambda j: (0, j + _NA // _BLK)),
    out_shape=jax.ShapeDtypeStruct((_D, _N), jnp.float32),
    input_output_aliases={1: 0},
)


def kernel(inputs, weight):
    xt = inputs.T
    wt2 = weight.T + weight.T
    w2col = jnp.sum(weight * weight, axis=1, keepdims=True)
    table_pad = jnp.pad(weight, ((0, 0), (0, _DP - _D)))

    quant, losses = [], []
    for h, n_tok in ((0, _NA), (1, _NB)):
        idx3, loss_sum = _tc_half[h](xt, wt2, w2col)
        idx_flat = idx3.reshape(n_tok)
        quant_pad = _make_sc_gather(n_tok)(idx_flat, table_pad)
        quant.append(quant_pad)
        losses.append((idx_flat, loss_sum))

    qt = _tp_first(quant[0])
    qt = _tp_second(quant[1], qt)
    quantized = qt.T
    idx_all = jnp.concatenate([l[0] for l in losses]).reshape(_N, 1)
    s = losses[0][1][0, 0] + losses[1][1][0, 0]
    q_latent_loss = s / (_N * _D)
    e_term = _COMMIT * q_latent_loss
    return quantized, q_latent_loss, e_term, idx_all


# speedup vs baseline: 1.0133x; 1.0133x over previous
import functools

import jax
import jax.numpy as jnp
from jax import lax
from jax.experimental import pallas as pl
from jax.experimental.pallas import tpu as pltpu
from jax.experimental.pallas import tpu_sc as plsc

_N = 18432
_K = 1024
_D = 64
_BLK = 3072
_COMMIT = 0.25
_NA = 6144
_NB = _N - _NA


def _tc_body(xt_ref, wt_ref, w2_ref, idx_ref, loss_ref):
    i = pl.program_id(0)
    xt = xt_ref[...]
    wt = wt_ref[...]
    mmt = lax.dot_general(wt, xt, (((0,), (0,)), ((), ())),
                          preferred_element_type=jnp.float32)
    z2 = jnp.sum(xt * xt, axis=0, keepdims=True)
    w2 = w2_ref[...]
    dist = (z2 + w2) - mmt
    minval = jnp.min(dist, axis=0, keepdims=True)
    idx = jnp.argmin(dist, axis=0).reshape(1, _BLK)
    idx_ref[...] = idx[None]

    @pl.when(i == 0)
    def _init():
        loss_ref[...] = jnp.zeros_like(loss_ref)

    loss_ref[...] = loss_ref[...] + jnp.sum(minval)


def _make_tc_call(n_tok, block_off):
    return pl.pallas_call(
        _tc_body,
        grid=(n_tok // _BLK,),
        in_specs=[
            pl.BlockSpec((_D, _BLK), lambda i: (0, i + block_off)),
            pl.BlockSpec((_D, _K), lambda i: (0, 0)),
            pl.BlockSpec((_K, 1), lambda i: (0, 0)),
        ],
        out_specs=[
            pl.BlockSpec((1, 1, _BLK), lambda i: (i, 0, 0)),
            pl.BlockSpec((1, 1), lambda i: (0, 0)),
        ],
        out_shape=[
            jax.ShapeDtypeStruct((n_tok // _BLK, 1, _BLK), jnp.int32),
            jax.ShapeDtypeStruct((1, 1), jnp.float32),
        ],
    )


_tc_half = [_make_tc_call(_NA, 0), _make_tc_call(_NB, _NA // _BLK)]

_DP = 128


@functools.cache
def _make_sc_gather(n_tok):
    info = plsc.get_sparse_core_info()
    nc, ns = info.num_cores, info.num_subcores
    nw = nc * ns
    bpw = n_tok // nw
    ch = 96
    n_ch = bpw // ch
    assert bpw % ch == 0 and n_tok % nw == 0
    mesh = plsc.VectorSubcoreMesh(core_axis_name="c", subcore_axis_name="s")

    @functools.partial(
        pl.kernel,
        mesh=mesh,
        out_type=jax.ShapeDtypeStruct((n_tok, _DP), jnp.float32),
        scratch_types=[
            pltpu.VMEM((bpw,), jnp.int32),
            pltpu.VMEM((bpw, _DP), jnp.float32),
            pltpu.SemaphoreType.DMA,
        ],
    )
    def gather_k(idx_hbm, table_hbm, out_hbm, idx_v, rows_v, sem):
        wid = lax.axis_index("s") * nc + lax.axis_index("c")
        base = wid * bpw
        pltpu.sync_copy(idx_hbm.at[pl.ds(base, bpw)], idx_v)
        copies = []
        for j in range(n_ch):
            copies.append(pltpu.async_copy(
                table_hbm.at[idx_v.at[pl.ds(j * ch, ch)]],
                rows_v.at[pl.ds(j * ch, ch)],
                sem,
            ))
        for cp in copies:
            cp.wait()
        pltpu.sync_copy(rows_v, out_hbm.at[pl.ds(base, bpw)])

    return gather_k


def _tp_body_first(q_ref, out_ref):
    out_ref[...] = q_ref[...][:, :_D].T


def _tp_body_second(q_ref, alias_ref, out_ref):
    del alias_ref
    out_ref[...] = q_ref[...][:, :_D].T


_tp_first = pl.pallas_call(
    _tp_body_first,
    grid=(_NA // _BLK,),
    in_specs=[pl.BlockSpec((_BLK, _DP), lambda j: (j, 0))],
    out_specs=pl.BlockSpec((_D, _BLK), lambda j: (0, j)),
    out_shape=jax.ShapeDtypeStruct((_D, _N), jnp.float32),
)

_tp_second = pl.pallas_call(
    _tp_body_second,
    grid=(_NB // _BLK,),
    in_specs=[
        pl.BlockSpec((_BLK, _DP), lambda j: (j, 0)),
        pl.BlockSpec(memory_space=pl.ANY),
    ],
    out_specs=pl.BlockSpec((_D, _BLK), lambda j: (0, j + _NA // _BLK)),
    out_shape=jax.ShapeDtypeStruct((_D, _N), jnp.float32),
    input_output_aliases={1: 0},
)


def kernel(inputs, weight):
    xt = inputs.T
    wt2 = weight.T + weight.T
    w2col = jnp.sum(weight * weight, axis=1, keepdims=True)
    table_pad = jnp.pad(weight, ((0, 0), (0, _DP - _D)))

    quant, losses = [], []
    for h, n_tok in ((0, _NA), (1, _NB)):
        idx3, loss_sum = _tc_half[h](xt, wt2, w2col)
        idx_flat = idx3.reshape(n_tok)
        quant_pad = _make_sc_gather(n_tok)(idx_flat, table_pad)
        quant.append(quant_pad)
        losses.append((idx_flat, loss_sum))

    qt = _tp_first(quant[0])
    qt = _tp_second(quant[1], qt)
    quantized = qt.T
    idx_all = jnp.concatenate([l[0] for l in losses]).reshape(_N, 1)
    s = losses[0][1][0, 0] + losses[1][1][0, 0]
    q_latent_loss = s / (_N * _D)
    e_term = _COMMIT * q_latent_loss
    return quantized, q_latent_loss, e_term, idx_all
